# baseline (speedup 1.0000x reference)
import jax
import jax.numpy as jnp
from jax.experimental import pallas as pl
from jax.experimental.pallas import tpu as pltpu

_NE = 1024
_NL = 8
_NF = 32
_K = 5
_E2 = 128
_S = 2


def _off(level):
    return 2 ** level - 1


def _bitrev(j, bits):
    r = 0
    for _ in range(bits):
        r = (r << 1) | (j & 1)
        j >>= 1
    return r


def _leaky(x):
    return jnp.maximum(x, 0.1 * x)


def _dot(a, b):
    return jnp.dot(a, b, preferred_element_type=jnp.float32)


def _mlp_multi(params, xs):
    n = len(params)
    for i, (Wt, b) in enumerate(params):
        xs = [_dot(Wt, x) + b for x in xs]
        if i < n - 1:
            xs = [_leaky(x) for x in xs]
    return xs


def _fold_sum(x, steps):
    for _ in range(steps):
        h = x.shape[1] // 2
        x = x[:, :h] + x[:, h:]
    return x


def _fold_max(x, steps):
    for _ in range(steps):
        h = x.shape[1] // 2
        x = jnp.maximum(x[:, :h], x[:, h:])
    return x


def _unpack(wrefs):
    groups = []
    idx = 0
    for _ in range(5):
        g = []
        for _ in range(4):
            g.append((wrefs[idx][...], wrefs[idx + 1][...]))
            idx += 2
        groups.append(g)
    return groups


def _lvl(x, L, E):
    return x[:, _off(L) * E:_off(L + 1) * E]


def _hlvs_multi(dyn_pre, dyn_post, xs, top, E):
    hs = _mlp_multi(dyn_pre, xs)
    ssums = [None] * len(hs)
    smaxs = [None] * len(hs)
    for L in range(top + 1):
        for i, h in enumerate(hs):
            seg = _lvl(h, L, E)
            s = _fold_sum(seg, L)
            m = _fold_max(seg, L)
            ssums[i] = s if ssums[i] is None else ssums[i] + s
            smaxs[i] = m if smaxs[i] is None else jnp.maximum(smaxs[i], m)
    cnt = float(2 ** (top + 1) - 1)
    W1, b1 = dyn_post[0]
    hs = [_leaky(_dot(W1[:, :_NF], ssums[i] / cnt)
                 + _dot(W1[:, _NF:], smaxs[i]) + b1)
          for i in range(len(hs))]
    return _mlp_multi(dyn_post[1:], hs)


def _gin_multi(params, xs, gfs, top, E):
    W1, b1 = params[0]
    xsums = []
    for x in xs:
        parts = [x[:, :E]]
        for L in range(1, top + 1):
            prev = _lvl(x, L - 1, E)
            parts.append(_lvl(x, L, E) + jnp.concatenate([prev, prev], axis=1))
        xsums.append(jnp.concatenate(parts, axis=1))
    gadds = []
    for gf in gfs:
        gterm = _dot(W1[:, _NF:], gf)
        gparts = [gterm + b1]
        t = 2.0 * gterm + b1
        for L in range(1, top + 1):
            t = jnp.concatenate([t, t], axis=1)
            gparts.append(t)
        gadds.append(jnp.concatenate(gparts, axis=1))
    hs = [_leaky(_dot(W1[:, :_NF], xsums[i]) + gadds[i])
          for i in range(len(xs))]
    return _mlp_multi(params[1:], hs)


def _split_multi(groups, xs, s, E):
    dyn_pre, dyn_post, branch_proj, conv_mlp, _ = groups
    gfs = _hlvs_multi(dyn_pre, dyn_post, xs, s, E)
    W1, b1 = branch_proj[0]
    gts = []
    for gf in gfs:
        gt = _dot(W1[:, _NF:], gf) + b1
        for _ in range(s):
            gt = jnp.concatenate([gt, gt], axis=1)
        gts.append(gt)
    hs = [_leaky(_dot(W1[:, :_NF], _lvl(xs[i], s, E)) + gts[i])
          for i in range(len(xs))]
    chs = _mlp_multi(branch_proj[1:], hs)
    out = []
    for i, ch in enumerate(chs):
        child = jnp.concatenate([ch[:_NF, :], ch[_NF:, :]], axis=1)
        out.append(jnp.concatenate([xs[i], child], axis=1))
    return _gin_multi(conv_mlp, out, gfs, s + 1, E)


def _p1_body(x0_ref, *refs):
    groups = _unpack(refs[:40])
    orefs = refs[40:]
    E = _NE
    xs = [x0_ref[...]]
    for s in range(_K):
        xs = _split_multi(groups, xs, s, E)
    xall = xs[0]
    for L in range(_K + 1):
        lv = _lvl(xall, L, E)
        if L == 0:
            orefs[L][...] = lv
        else:
            orefs[L][...] = jnp.concatenate(
                [lv[:, bi * E:(bi + 1) * E] for bi in range(2 ** L)], axis=0)


def _p2_body(*refs):
    lrefs = refs[:_K + 1]
    groups = _unpack(refs[_K + 1:_K + 41])
    orefs = refs[_K + 41:]
    E = _E2
    xs = []
    for i in range(_S):
        parts = []
        for L in range(_K + 1):
            a = lrefs[L][...]
            parts.extend(a[bi * _NF:(bi + 1) * _NF, i * E:(i + 1) * E]
                         for bi in range(2 ** L))
        xs.append(jnp.concatenate(parts, axis=1))
    for s in range(_K, _NL - 1):
        xs = _split_multi(groups, xs, s, E)
    dyn_pre, dyn_post, _, _, pp_conv_mlp = groups
    for _ in range(2):
        gfs = _hlvs_multi(dyn_pre, dyn_post, xs, _NL - 1, E)
        xs = _gin_multi(pp_conv_mlp, xs, gfs, _NL - 1, E)
    for i in range(_S):
        y = xs[i][:3, :]
        for L in range(_NL):
            yl = _lvl(y, L, E)
            blocks = [yl[:, _bitrev(j, L) * E:(_bitrev(j, L) + 1) * E]
                      for j in range(2 ** L)]
            orefs[L][:, i * E * 2 ** L:(i + 1) * E * 2 ** L] = (
                blocks[0] if L == 0 else jnp.concatenate(blocks, axis=1))


def kernel(random_vector, dyn_pre, dyn_post, branch_proj, conv_mlp, pp_conv_mlp):
    x0 = random_vector.reshape(_NE, _NF).T
    wflat = []
    for g in (dyn_pre, dyn_post, branch_proj, conv_mlp, pp_conv_mlp):
        for W, b in g:
            wflat.append(W.T)
            wflat.append(b.reshape(-1, 1))
    wspecs = [pl.BlockSpec(w.shape, lambda b: (0, 0)) for w in wflat]

    p1_out_shapes = [jax.ShapeDtypeStruct((2 ** L * _NF, _NE), jnp.float32)
                     for L in range(_K + 1)]
    p1_out_specs = [pl.BlockSpec((2 ** L * _NF, _NE), lambda b: (0, 0))
                    for L in range(_K + 1)]
    levels = pl.pallas_call(
        _p1_body,
        grid=(1,),
        in_specs=[pl.BlockSpec((_NF, _NE), lambda b: (0, 0))] + wspecs,
        out_specs=p1_out_specs,
        out_shape=p1_out_shapes,
        compiler_params=pltpu.CompilerParams(
            dimension_semantics=("parallel",),
            vmem_limit_bytes=100 * 1024 * 1024),
    )(x0, *wflat)

    EB = _E2 * _S
    nblk = _NE // EB
    p2_in_specs = [pl.BlockSpec((2 ** L * _NF, EB), lambda b: (0, b))
                   for L in range(_K + 1)] + wspecs
    out_shapes = [jax.ShapeDtypeStruct((3, _NE * 2 ** L), jnp.float32)
                  for L in range(_NL)]
    out_specs = [pl.BlockSpec((3, EB * 2 ** L), lambda b: (0, b))
                 for L in range(_NL)]
    outs = pl.pallas_call(
        _p2_body,
        grid=(nblk,),
        in_specs=p2_in_specs,
        out_specs=out_specs,
        out_shape=out_shapes,
        compiler_params=pltpu.CompilerParams(
            dimension_semantics=("parallel",),
            vmem_limit_bytes=100 * 1024 * 1024),
    )(*levels, *wflat)

    res = []
    for L, o in enumerate(outs):
        o4 = o.reshape(3, _NE // _E2, 2 ** L, _E2)
        res.append(o4.transpose(1, 3, 2, 0).reshape(_NE * 2 ** L, 3))
    return jnp.concatenate(res, axis=0)


# speedup vs baseline: 1.1715x; 1.1715x over previous
import jax
import jax.numpy as jnp
from jax.experimental import pallas as pl
from jax.experimental.pallas import tpu as pltpu

_NE = 1024
_NL = 8
_NF = 32
_K = 5
_E2 = 128
_S = 1


def _off(level):
    return 2 ** level - 1


def _bitrev(j, bits):
    r = 0
    for _ in range(bits):
        r = (r << 1) | (j & 1)
        j >>= 1
    return r


def _leaky(x):
    return jnp.maximum(x, 0.1 * x)


def _dot(a, b):
    return jnp.dot(a, b, preferred_element_type=jnp.float32)


def _mlp_multi(params, xs):
    n = len(params)
    for i, (Wt, b) in enumerate(params):
        xs = [_dot(Wt, x) + b for x in xs]
        if i < n - 1:
            xs = [_leaky(x) for x in xs]
    return xs


def _fold_sum(x, steps):
    for _ in range(steps):
        h = x.shape[1] // 2
        x = x[:, :h] + x[:, h:]
    return x


def _fold_max(x, steps):
    for _ in range(steps):
        h = x.shape[1] // 2
        x = jnp.maximum(x[:, :h], x[:, h:])
    return x


def _unpack(wrefs):
    groups = []
    idx = 0
    for _ in range(5):
        g = []
        for _ in range(4):
            g.append((wrefs[idx][...], wrefs[idx + 1][...]))
            idx += 2
        groups.append(g)
    return groups


def _lvl(x, L, E):
    return x[:, _off(L) * E:_off(L + 1) * E]


def _hlvs_multi(dyn_pre, dyn_post, xs, top, E):
    hs = _mlp_multi(dyn_pre, xs)
    ssums = [None] * len(hs)
    smaxs = [None] * len(hs)
    for L in range(top + 1):
        for i, h in enumerate(hs):
            seg = _lvl(h, L, E)
            s = _fold_sum(seg, L)
            m = _fold_max(seg, L)
            ssums[i] = s if ssums[i] is None else ssums[i] + s
            smaxs[i] = m if smaxs[i] is None else jnp.maximum(smaxs[i], m)
    cnt = float(2 ** (top + 1) - 1)
    W1, b1 = dyn_post[0]
    hs = [_leaky(_dot(W1[:, :_NF], ssums[i] / cnt)
                 + _dot(W1[:, _NF:], smaxs[i]) + b1)
          for i in range(len(hs))]
    return _mlp_multi(dyn_post[1:], hs)


def _gin_multi(params, xs, gfs, top, E):
    W1, b1 = params[0]
    xsums = []
    for x in xs:
        parts = [x[:, :E]]
        for L in range(1, top + 1):
            prev = _lvl(x, L - 1, E)
            parts.append(_lvl(x, L, E) + jnp.concatenate([prev, prev], axis=1))
        xsums.append(jnp.concatenate(parts, axis=1))
    gadds = []
    for gf in gfs:
        gterm = _dot(W1[:, _NF:], gf)
        gparts = [gterm + b1]
        t = 2.0 * gterm + b1
        for L in range(1, top + 1):
            t = jnp.concatenate([t, t], axis=1)
            gparts.append(t)
        gadds.append(jnp.concatenate(gparts, axis=1))
    hs = [_leaky(_dot(W1[:, :_NF], xsums[i]) + gadds[i])
          for i in range(len(xs))]
    return _mlp_multi(params[1:], hs)


def _split_multi(groups, xs, s, E):
    dyn_pre, dyn_post, branch_proj, conv_mlp, _ = groups
    gfs = _hlvs_multi(dyn_pre, dyn_post, xs, s, E)
    W1, b1 = branch_proj[0]
    gts = []
    for gf in gfs:
        gt = _dot(W1[:, _NF:], gf) + b1
        for _ in range(s):
            gt = jnp.concatenate([gt, gt], axis=1)
        gts.append(gt)
    hs = [_leaky(_dot(W1[:, :_NF], _lvl(xs[i], s, E)) + gts[i])
          for i in range(len(xs))]
    chs = _mlp_multi(branch_proj[1:], hs)
    out = []
    for i, ch in enumerate(chs):
        child = jnp.concatenate([ch[:_NF, :], ch[_NF:, :]], axis=1)
        out.append(jnp.concatenate([xs[i], child], axis=1))
    return _gin_multi(conv_mlp, out, gfs, s + 1, E)


def _p1_body(x0_ref, *refs):
    groups = _unpack(refs[:40])
    orefs = refs[40:]
    E = _NE
    xs = [x0_ref[...]]
    for s in range(_K):
        xs = _split_multi(groups, xs, s, E)
    xall = xs[0]
    for L in range(_K + 1):
        lv = _lvl(xall, L, E)
        if L == 0:
            orefs[L][...] = lv
        else:
            orefs[L][...] = jnp.concatenate(
                [lv[:, bi * E:(bi + 1) * E] for bi in range(2 ** L)], axis=0)


def _p2_body(*refs):
    lrefs = refs[:_K + 1]
    groups = _unpack(refs[_K + 1:_K + 41])
    orefs = refs[_K + 41:]
    E = _E2
    xs = []
    for i in range(_S):
        parts = []
        for L in range(_K + 1):
            a = lrefs[L][...]
            parts.extend(a[bi * _NF:(bi + 1) * _NF, i * E:(i + 1) * E]
                         for bi in range(2 ** L))
        xs.append(jnp.concatenate(parts, axis=1))
    for s in range(_K, _NL - 1):
        xs = _split_multi(groups, xs, s, E)
    dyn_pre, dyn_post, _, _, pp_conv_mlp = groups
    for _ in range(2):
        gfs = _hlvs_multi(dyn_pre, dyn_post, xs, _NL - 1, E)
        xs = _gin_multi(pp_conv_mlp, xs, gfs, _NL - 1, E)
    for i in range(_S):
        y = xs[i][:3, :]
        for L in range(_NL):
            yl = _lvl(y, L, E)
            blocks = [yl[:, _bitrev(j, L) * E:(_bitrev(j, L) + 1) * E]
                      for j in range(2 ** L)]
            orefs[L][:, i * E * 2 ** L:(i + 1) * E * 2 ** L] = (
                blocks[0] if L == 0 else jnp.concatenate(blocks, axis=1))


def kernel(random_vector, dyn_pre, dyn_post, branch_proj, conv_mlp, pp_conv_mlp):
    x0 = random_vector.reshape(_NE, _NF).T
    wflat = []
    for g in (dyn_pre, dyn_post, branch_proj, conv_mlp, pp_conv_mlp):
        for W, b in g:
            wflat.append(W.T)
            wflat.append(b.reshape(-1, 1))
    wspecs = [pl.BlockSpec(w.shape, lambda b: (0, 0)) for w in wflat]

    p1_out_shapes = [jax.ShapeDtypeStruct((2 ** L * _NF, _NE), jnp.float32)
                     for L in range(_K + 1)]
    p1_out_specs = [pl.BlockSpec((2 ** L * _NF, _NE), lambda b: (0, 0))
                    for L in range(_K + 1)]
    levels = pl.pallas_call(
        _p1_body,
        grid=(1,),
        in_specs=[pl.BlockSpec((_NF, _NE), lambda b: (0, 0))] + wspecs,
        out_specs=p1_out_specs,
        out_shape=p1_out_shapes,
        compiler_params=pltpu.CompilerParams(
            dimension_semantics=("parallel",),
            vmem_limit_bytes=100 * 1024 * 1024),
    )(x0, *wflat)

    EB = _E2 * _S
    nblk = _NE // EB
    p2_in_specs = [pl.BlockSpec((2 ** L * _NF, EB), lambda b: (0, b))
                   for L in range(_K + 1)] + wspecs
    out_shapes = [jax.ShapeDtypeStruct((3, _NE * 2 ** L), jnp.float32)
                  for L in range(_NL)]
    out_specs = [pl.BlockSpec((3, EB * 2 ** L), lambda b: (0, b))
                 for L in range(_NL)]
    outs = pl.pallas_call(
        _p2_body,
        grid=(nblk,),
        in_specs=p2_in_specs,
        out_specs=out_specs,
        out_shape=out_shapes,
        compiler_params=pltpu.CompilerParams(
            dimension_semantics=("parallel",),
            vmem_limit_bytes=100 * 1024 * 1024),
    )(*levels, *wflat)

    res = []
    for L, o in enumerate(outs):
        o4 = o.reshape(3, _NE // _E2, 2 ** L, _E2)
        res.append(o4.transpose(1, 3, 2, 0).reshape(_NE * 2 ** L, 3))
    return jnp.concatenate(res, axis=0)
